# Initial kernel scaffold; baseline (speedup 1.0000x reference)
import jax
import jax.numpy as jnp
from jax.experimental import pallas as pl


def kernel(activation, ema_activation, pseudo_label, queue_list):
    raise NotImplementedError("write your pallas kernel here")


# speedup vs baseline: 1.3884x; 1.3884x over previous
import functools

import jax
import jax.numpy as jnp
from jax.experimental import pallas as pl
from jax.experimental.pallas import tpu as pltpu

_PROJ_DIM = 128
_CLASS_NUM = 7
_QUEUE_SIZE = 168
_TEMPERATURE = 0.5
_QC = _QUEUE_SIZE * _CLASS_NUM

_BLOCK_B = 512


def _pgc_kernel(act_ref, ema_ref, lbl_ref, queue_ref, out_ref):
    i = pl.program_id(0)

    act = act_ref[...]
    ema = ema_ref[...]

    f_norm = jnp.maximum(jnp.sqrt(jnp.sum(act * act, axis=1, keepdims=True)), 1e-12)
    e_norm = jnp.maximum(jnp.sqrt(jnp.sum(ema * ema, axis=1, keepdims=True)), 1e-12)
    feature = act / f_norm
    ema_feature = ema / e_norm

    l_pos = jnp.sum(feature * ema_feature, axis=1, keepdims=True)

    sim = jax.lax.dot_general(
        feature, queue_ref[...],
        dimension_numbers=(((1,), (1,)), ((), ())),
        preferred_element_type=jnp.float32,
    )

    exp_sim = jnp.exp(sim * (1.0 / _TEMPERATURE))
    total = jnp.sum(exp_sim, axis=1, keepdims=True)
    denom = l_pos + total

    col_class = jax.lax.broadcasted_iota(jnp.int32, sim.shape, 1) // _QUEUE_SIZE
    mask = col_class == lbl_ref[...]

    term_queue = -jnp.log(exp_sim / denom + 1e-8)
    pos_log_sum = jnp.sum(jnp.where(mask, term_queue, 0.0), axis=1, keepdims=True)

    term_ema = -jnp.log(l_pos / denom + 1e-8)
    contrast = (term_ema + pos_log_sum) * (1.0 / (_QUEUE_SIZE + 1))

    partial = jnp.sum(contrast, axis=0, keepdims=True) * (
        1.0 / (contrast.shape[0] * pl.num_programs(0))
    )

    @pl.when(i == 0)
    def _init():
        out_ref[...] = jnp.zeros_like(out_ref)

    out_ref[...] += partial


@functools.partial(jax.jit, static_argnames=())
def kernel(activation, ema_activation, pseudo_label, queue_list):
    batch = activation.shape[0]
    labels = pseudo_label.reshape(batch, 1).astype(jnp.int32)
    grid = (batch // _BLOCK_B,)

    out = pl.pallas_call(
        _pgc_kernel,
        grid=grid,
        in_specs=[
            pl.BlockSpec((_BLOCK_B, _PROJ_DIM), lambda i: (i, 0)),
            pl.BlockSpec((_BLOCK_B, _PROJ_DIM), lambda i: (i, 0)),
            pl.BlockSpec((_BLOCK_B, 1), lambda i: (i, 0)),
            pl.BlockSpec((_QC, _PROJ_DIM), lambda i: (0, 0)),
        ],
        out_specs=pl.BlockSpec((1, 1), lambda i: (0, 0)),
        out_shape=jax.ShapeDtypeStruct((1, 1), jnp.float32),
    )(activation, ema_activation, labels, queue_list)

    return out[0, 0]


# speedup vs baseline: 1.6569x; 1.1935x over previous
import functools

import jax
import jax.numpy as jnp
from jax.experimental import pallas as pl
from jax.experimental.pallas import tpu as pltpu

_PROJ_DIM = 128
_CLASS_NUM = 7
_QUEUE_SIZE = 168
_TEMPERATURE = 0.5
_QC = _QUEUE_SIZE * _CLASS_NUM

_BLOCK_B = 512


def _pgc_kernel(act_ref, ema_ref, lbl_ref, queue_ref, out_ref):
    i = pl.program_id(0)

    act = act_ref[...]
    ema = ema_ref[...]

    f_norm = jnp.maximum(jnp.sqrt(jnp.sum(act * act, axis=1, keepdims=True)), 1e-12)
    e_norm = jnp.maximum(jnp.sqrt(jnp.sum(ema * ema, axis=1, keepdims=True)), 1e-12)
    feature = act / f_norm
    ema_feature = ema / e_norm

    l_pos = jnp.sum(feature * ema_feature, axis=1, keepdims=True)

    sim = jax.lax.dot_general(
        feature, queue_ref[...],
        dimension_numbers=(((1,), (1,)), ((), ())),
        preferred_element_type=jnp.float32,
    )

    exp_sim = jnp.exp(sim * (1.0 / _TEMPERATURE))
    total = jnp.sum(exp_sim, axis=1, keepdims=True)
    denom = l_pos + total

    col_class = jax.lax.broadcasted_iota(jnp.int32, sim.shape, 1) // _QUEUE_SIZE
    mask = col_class == lbl_ref[...]

    masked_sim = jnp.sum(jnp.where(mask, sim, 0.0), axis=1, keepdims=True)
    pos_log_sum = _QUEUE_SIZE * jnp.log(denom) - masked_sim * (1.0 / _TEMPERATURE)

    term_ema = -jnp.log(l_pos / denom + 1e-8)
    contrast = (term_ema + pos_log_sum) * (1.0 / (_QUEUE_SIZE + 1))

    partial = jnp.sum(contrast, axis=0, keepdims=True) * (
        1.0 / (contrast.shape[0] * pl.num_programs(0))
    )

    @pl.when(i == 0)
    def _init():
        out_ref[...] = jnp.zeros_like(out_ref)

    out_ref[...] += partial


@functools.partial(jax.jit, static_argnames=())
def kernel(activation, ema_activation, pseudo_label, queue_list):
    batch = activation.shape[0]
    labels = pseudo_label.reshape(batch, 1).astype(jnp.int32)
    grid = (batch // _BLOCK_B,)

    out = pl.pallas_call(
        _pgc_kernel,
        grid=grid,
        in_specs=[
            pl.BlockSpec((_BLOCK_B, _PROJ_DIM), lambda i: (i, 0)),
            pl.BlockSpec((_BLOCK_B, _PROJ_DIM), lambda i: (i, 0)),
            pl.BlockSpec((_BLOCK_B, 1), lambda i: (i, 0)),
            pl.BlockSpec((_QC, _PROJ_DIM), lambda i: (0, 0)),
        ],
        out_specs=pl.BlockSpec((1, 1), lambda i: (0, 0)),
        out_shape=jax.ShapeDtypeStruct((1, 1), jnp.float32),
    )(activation, ema_activation, labels, queue_list)

    return out[0, 0]
